# baseline (speedup 1.0000x reference)
import functools

import jax
import jax.numpy as jnp
from jax import lax
from jax.experimental import pallas as pl
from jax.experimental.pallas import tpu as pltpu
from jax.experimental.pallas import tpu_sc as plsc

NN = 10000
NE = 20000
DIN = 768
HID = 128
DOUT = 128
NTY = 4
NB = 3

TE = 256
EP = 22528
NT = EP // TE
ACC_ROWS = 10240
LNC = 400
FNC = 2048

NC = 2
NS = 16
NW = NC * NS
CH = 128
RPW = EP // NW
GCH = RPW // CH
RW4 = 2 * EP // NW
SCH4 = RW4 // CH
RPT = ACC_ROWS // NS


HD2 = DIN // 2


def _bf16_bits(v):
    b = lax.bitcast_convert_type(v, jnp.uint32)
    rnd = ((b >> 16) & 1) + jnp.uint32(0x7FFF)
    return (b + rnd) >> 16


def _pack_halves(x):
    lo = _bf16_bits(x[:, :HD2])
    hi = _bf16_bits(x[:, HD2:])
    return lax.bitcast_convert_type(lo | (hi << 16), jnp.int32)


def _unpack_halves(w):
    wu = lax.bitcast_convert_type(w, jnp.uint32)
    lo = lax.bitcast_convert_type(wu << 16, jnp.float32)
    hi = lax.bitcast_convert_type(wu & jnp.uint32(0xFFFF0000), jnp.float32)
    return lo, hi


def _k1_body(h_ref, g_ref, b_ref, w6_ref, hnb_ref, y6n_ref):
    x = h_ref[:]
    mu = jnp.mean(x, axis=-1, keepdims=True)
    var = jnp.mean((x - mu) ** 2, axis=-1, keepdims=True)
    ln = (x - mu) / jnp.sqrt(var + 1e-5) * g_ref[:] + b_ref[:]
    hnb_ref[:] = _pack_halves(ln)
    y6n_ref[:] = _pack_halves(jnp.dot(ln.astype(jnp.bfloat16), w6_ref[:],
                                      preferred_element_type=jnp.float32))


GC = 64
NGC = RPW // GC


@functools.cache
def _sc_gather_kernel():
    mesh = plsc.VectorSubcoreMesh(core_axis_name="c", subcore_axis_name="s")

    @functools.partial(
        pl.kernel,
        out_type=(jax.ShapeDtypeStruct((EP, HD2), jnp.int32),
                  jax.ShapeDtypeStruct((EP, HD2), jnp.int32)),
        mesh=mesh,
        scratch_types=[pltpu.VMEM((RPW,), jnp.int32),
                       pltpu.VMEM((GC, HD2), jnp.int32),
                       pltpu.VMEM((GC, HD2), jnp.int32),
                       pltpu.SemaphoreType.DMA,
                       pltpu.SemaphoreType.DMA],
    )
    def _k2_gather(y6n_h, hnb_h, srcp_h, dstp_h, g6_o, gd_o, idxf, rows0,
                   rows1, sem0, sem1):
        wid = lax.axis_index("s") * NC + lax.axis_index("c")
        base = wid * RPW
        bufs = (rows0, rows1)
        sems = (sem0, sem1)
        for tbl, idxarr, out in ((y6n_h, srcp_h, g6_o), (hnb_h, dstp_h, gd_o)):
            pltpu.sync_copy(idxarr.at[pl.ds(base, RPW)], idxf)
            cps = [None] * NGC
            cps[0] = pltpu.async_copy(
                tbl.at[idxf.at[pl.ds(0, GC)]], bufs[0], sems[0])
            for c in range(1, NGC):
                cps[c] = pltpu.async_copy(
                    tbl.at[idxf.at[pl.ds(c * GC, GC)]], bufs[c % 2],
                    sems[c % 2])
                cps[c - 1].wait()
                pltpu.sync_copy(bufs[(c - 1) % 2],
                                out.at[pl.ds(base + (c - 1) * GC, GC)])
            cps[NGC - 1].wait()
            pltpu.sync_copy(bufs[(NGC - 1) % 2],
                            out.at[pl.ds(base + (NGC - 1) * GC, GC)])

    return _k2_gather


def _k3_body(tt_s, g6_ref, gd_ref, et_ref, ht_ref, tc_ref, uc_ref,
             wtl_ref, wth_ref, w1_ref, w2_ref, p_ref):
    et = et_ref[:]
    ht = ht_ref[:]
    zero = jnp.zeros((TE, NB), jnp.float32)

    def sel_coeff(table_ref):
        c = zero
        for t in range(NTY):
            c = c + jnp.where(et == t, table_ref[t:t + 1, :], 0.0)
        return c

    c_text = sel_coeff(tc_ref)
    c_user = sel_coeff(uc_ref)
    is_h = ht > 0
    c_edge = jnp.concatenate(
        [jnp.where(is_h, c_text, 0.0), jnp.where(is_h, 0.0, c_user)], axis=1)

    y6lo, y6hi = _unpack_halves(g6_ref[:])
    ef = jnp.zeros((TE, HID), jnp.float32)
    for bi in range(NB):
        ef = ef + c_edge[:, bi:bi + 1] * y6lo[:, bi * HID:(bi + 1) * HID]
    for bi in range(NB, 2 * NB):
        ef = ef + (c_edge[:, bi:bi + 1]
                   * y6hi[:, (bi - NB) * HID:(bi - NB + 1) * HID])
    ef = jnp.maximum(ef, 0.0)

    gdlo, gdhi = _unpack_halves(gd_ref[:])
    yd = jnp.maximum(
        jnp.dot(gdlo.astype(jnp.bfloat16), wtl_ref[0],
                preferred_element_type=jnp.float32)
        + jnp.dot(gdhi.astype(jnp.bfloat16), wth_ref[0],
                  preferred_element_type=jnp.float32), 0.0)

    p_ref[0] = jnp.dot(ef.astype(jnp.bfloat16), w1_ref[:],
                       preferred_element_type=jnp.float32)
    p_ref[1] = jnp.dot(yd.astype(jnp.bfloat16), w2_ref[:],
                       preferred_element_type=jnp.float32)


@functools.cache
def _sc_scatter_kernel():
    mesh = plsc.VectorSubcoreMesh(core_axis_name="c", subcore_axis_name="s")

    @functools.partial(
        pl.kernel,
        out_type=jax.ShapeDtypeStruct((NC * ACC_ROWS, HID), jnp.float32),
        mesh=mesh,
        scratch_types=[pltpu.VMEM((CH,), jnp.int32),
                       pltpu.VMEM((CH, HID), jnp.float32),
                       pltpu.MemorySpace.VMEM_SHARED((ACC_ROWS, HID),
                                                     jnp.float32)],
    )
    def _k4_scatter(p_h, i_h, zin_h, out_o, idx_v, rows_v, shared):
        cid = lax.axis_index("c")
        sid = lax.axis_index("s")
        wid = sid * NC + cid
        pltpu.sync_copy(zin_h.at[pl.ds(sid * RPT, RPT)],
                        shared.at[pl.ds(sid * RPT, RPT)])
        plsc.subcore_barrier()
        base = wid * RW4
        for c in range(SCH4):
            off = base + c * CH
            pltpu.sync_copy(i_h.at[pl.ds(off, CH)], idx_v)
            pltpu.sync_copy(p_h.at[pl.ds(off, CH)], rows_v)
            pltpu.sync_copy(rows_v, shared.at[idx_v], add=True)
        plsc.subcore_barrier()
        pltpu.sync_copy(shared.at[pl.ds(sid * RPT, RPT)],
                        out_o.at[pl.ds(cid * ACC_ROWS + sid * RPT, RPT)])

    return _k4_scatter


def _k5_body(pa_ref, lb_ref, out_ref):
    z = jnp.maximum(pa_ref[0] + pa_ref[1] + lb_ref[:], 0.0)
    zn = jnp.sqrt(jnp.sum(z * z, axis=-1, keepdims=True))
    zn = jnp.where(zn == 0.0, 1.0, zn)
    out_ref[:] = z / zn


def kernel(h, edge_index, etype, hete, ln_g, ln_b, text_Wb, text_c,
           user_Wb, user_c, dst_Wb, dst_c, lin_W, lin_b):
    src = edge_index[0].astype(jnp.int32)
    dst = edge_index[1].astype(jnp.int32)

    ish = (hete > 0).astype(jnp.int32)
    ch = jnp.cumsum(ish)
    nh = ch[-1]
    cnh = jnp.arange(1, NE + 1, dtype=jnp.int32) - ch
    pos = jnp.where(ish > 0, ch - 1, nh + cnh - 1)
    tgt = dst[pos]

    ety = etype.astype(jnp.int32)
    oh = (ety[:, None] == jnp.arange(NTY, dtype=jnp.int32)[None, :])
    csum = jnp.cumsum(oh.astype(jnp.int32), axis=0)
    cnt = csum[-1]
    cap = ((cnt + TE - 1) // TE) * TE
    offs = jnp.concatenate([jnp.zeros((1,), jnp.int32),
                            jnp.cumsum(cap)[:-1].astype(jnp.int32)])
    rank = jnp.take_along_axis(csum, ety[:, None], axis=1)[:, 0] - 1
    q = offs[ety] + rank

    hete_i = (hete > 0).astype(jnp.int32)
    src_p = jnp.zeros((EP,), jnp.int32).at[q].set(src)
    dst_p = jnp.zeros((EP,), jnp.int32).at[q].set(dst)
    tgt_q = jnp.full((EP,), NN, jnp.int32).at[q].set(tgt)
    sct_q = jnp.full((EP,), NN, jnp.int32).at[q].set(dst)
    scat_i = jnp.concatenate([tgt_q, sct_q])
    ht_p = jnp.zeros((EP,), jnp.int32).at[q].set(hete_i).reshape(EP, 1)

    tile_start = jnp.arange(NT, dtype=jnp.int32) * TE
    grp_end = offs + cap
    ttype = jnp.minimum(
        jnp.sum((tile_start[:, None] >= grp_end[None, :]).astype(jnp.int32),
                axis=1), NTY - 1)
    et_p = jnp.repeat(ttype, TE).reshape(EP, 1)

    w6 = jnp.concatenate([
        jnp.moveaxis(text_Wb, 0, 1).reshape(DIN, NB * HID),
        jnp.moveaxis(user_Wb, 0, 1).reshape(DIN, NB * HID)],
        axis=1).astype(jnp.bfloat16)
    wt = jnp.einsum('tb,bio->tio', dst_c, dst_Wb).astype(jnp.bfloat16)
    wtl = wt[:, :HD2, :]
    wth = wt[:, HD2:, :]
    w1 = lin_W[:HID].astype(jnp.bfloat16)
    w2 = lin_W[HID:].astype(jnp.bfloat16)

    hnb, y6n = pl.pallas_call(
        _k1_body,
        grid=(NN // LNC,),
        in_specs=[
            pl.BlockSpec((LNC, DIN), lambda i: (i, 0)),
            pl.BlockSpec((1, DIN), lambda i: (0, 0)),
            pl.BlockSpec((1, DIN), lambda i: (0, 0)),
            pl.BlockSpec((DIN, 2 * NB * HID), lambda i: (0, 0)),
        ],
        out_specs=[pl.BlockSpec((LNC, HD2), lambda i: (i, 0)),
                   pl.BlockSpec((LNC, HD2), lambda i: (i, 0))],
        out_shape=[jax.ShapeDtypeStruct((NN, HD2), jnp.int32),
                   jax.ShapeDtypeStruct((NN, HD2), jnp.int32)],
    )(h, ln_g.reshape(1, DIN), ln_b.reshape(1, DIN), w6)

    g6, gd = _sc_gather_kernel()(y6n, hnb, src_p, dst_p)

    k3_grid = pltpu.PrefetchScalarGridSpec(
        num_scalar_prefetch=1,
        grid=(NT,),
        in_specs=[
            pl.BlockSpec((TE, HD2), lambda i, tt: (i, 0)),
            pl.BlockSpec((TE, HD2), lambda i, tt: (i, 0)),
            pl.BlockSpec((TE, 1), lambda i, tt: (i, 0)),
            pl.BlockSpec((TE, 1), lambda i, tt: (i, 0)),
            pl.BlockSpec((NTY, NB), lambda i, tt: (0, 0)),
            pl.BlockSpec((NTY, NB), lambda i, tt: (0, 0)),
            pl.BlockSpec((1, HD2, DIN), lambda i, tt: (tt[i], 0, 0)),
            pl.BlockSpec((1, HD2, DIN), lambda i, tt: (tt[i], 0, 0)),
            pl.BlockSpec((HID, DOUT), lambda i, tt: (0, 0)),
            pl.BlockSpec((DIN, DOUT), lambda i, tt: (0, 0)),
        ],
        out_specs=pl.BlockSpec((2, TE, HID), lambda i, tt: (0, i, 0)),
    )
    p = pl.pallas_call(
        _k3_body,
        grid_spec=k3_grid,
        out_shape=jax.ShapeDtypeStruct((2, EP, HID), jnp.float32),
    )(ttype, g6, gd, et_p, ht_p, text_c, user_c, wtl, wth, w1, w2)

    zin = jnp.zeros((ACC_ROWS, HID), jnp.float32)
    part = _sc_scatter_kernel()(p.reshape(2 * EP, HID), scat_i, zin)
    part = part.reshape(NC, ACC_ROWS, HID)

    outp = pl.pallas_call(
        _k5_body,
        grid=(ACC_ROWS // FNC,),
        in_specs=[
            pl.BlockSpec((NC, FNC, HID), lambda i: (0, i, 0)),
            pl.BlockSpec((1, DOUT), lambda i: (0, 0)),
        ],
        out_specs=pl.BlockSpec((FNC, DOUT), lambda i: (i, 0)),
        out_shape=jax.ShapeDtypeStruct((ACC_ROWS, DOUT), jnp.float32),
    )(part, lin_b.reshape(1, DOUT))
    return outp[:NN]


# speedup vs baseline: 1.9387x; 1.9387x over previous
import functools

import jax
import jax.numpy as jnp
from jax import lax
from jax.experimental import pallas as pl
from jax.experimental.pallas import tpu as pltpu
from jax.experimental.pallas import tpu_sc as plsc

NN = 10000
NE = 20000
DIN = 768
HID = 128
DOUT = 128
NTY = 4
NB = 3

TE = 256
EP = 20480
NT = EP // TE
ACC_ROWS = 10240
LNC = 400
FNC = 2048

NC = 2
NS = 16
NW = NC * NS
CH = 128
RPW = EP // NW
GCH = RPW // CH
RW4 = 2 * EP // NW
SCH4 = RW4 // CH
RPT = ACC_ROWS // NS


HD2 = DIN // 2


def _bf16_bits(v):
    b = lax.bitcast_convert_type(v, jnp.uint32)
    rnd = ((b >> 16) & 1) + jnp.uint32(0x7FFF)
    return (b + rnd) >> 16


def _pack_halves(x):
    lo = _bf16_bits(x[:, :HD2])
    hi = _bf16_bits(x[:, HD2:])
    return lax.bitcast_convert_type(lo | (hi << 16), jnp.int32)


def _unpack_halves(w):
    wu = lax.bitcast_convert_type(w, jnp.uint32)
    lo = lax.bitcast_convert_type(wu << 16, jnp.float32)
    hi = lax.bitcast_convert_type(wu & jnp.uint32(0xFFFF0000), jnp.float32)
    return lo, hi


def _k1_body(h_ref, g_ref, b_ref, w6_ref, hnb_ref, y6n_ref):
    x = h_ref[:]
    mu = jnp.mean(x, axis=-1, keepdims=True)
    var = jnp.mean((x - mu) ** 2, axis=-1, keepdims=True)
    ln = (x - mu) / jnp.sqrt(var + 1e-5) * g_ref[:] + b_ref[:]
    hnb_ref[:] = _pack_halves(ln)
    y6n_ref[:] = _pack_halves(jnp.dot(ln.astype(jnp.bfloat16), w6_ref[:],
                                      preferred_element_type=jnp.float32))


GC = 64
NGC = RPW // GC
NBUF = 4


@functools.cache
def _sc_gather_kernel():
    mesh = plsc.VectorSubcoreMesh(core_axis_name="c", subcore_axis_name="s")

    @functools.partial(
        pl.kernel,
        out_type=(jax.ShapeDtypeStruct((EP, HD2), jnp.int32),
                  jax.ShapeDtypeStruct((EP, HD2), jnp.int32)),
        mesh=mesh,
        scratch_types=[pltpu.VMEM((RPW,), jnp.int32)]
                      + [pltpu.VMEM((GC, HD2), jnp.int32)] * NBUF
                      + [pltpu.SemaphoreType.DMA] * NBUF,
    )
    def _k2_gather(y6n_h, hnb_h, srcp_h, dstp_h, g6_o, gd_o, idxf, *bufsems):
        bufs = bufsems[:NBUF]
        sems = bufsems[NBUF:]
        wid = lax.axis_index("s") * NC + lax.axis_index("c")
        base = wid * RPW
        for tbl, idxarr, out in ((y6n_h, srcp_h, g6_o), (hnb_h, dstp_h, gd_o)):
            pltpu.sync_copy(idxarr.at[pl.ds(base, RPW)], idxf)
            cps = [None] * NGC
            for c in range(NBUF - 1):
                cps[c] = pltpu.async_copy(
                    tbl.at[idxf.at[pl.ds(c * GC, GC)]], bufs[c % NBUF],
                    sems[c % NBUF])
            for c in range(NGC):
                nc = c + NBUF - 1
                if nc < NGC:
                    cps[nc] = pltpu.async_copy(
                        tbl.at[idxf.at[pl.ds(nc * GC, GC)]], bufs[nc % NBUF],
                        sems[nc % NBUF])
                cps[c].wait()
                pltpu.sync_copy(bufs[c % NBUF],
                                out.at[pl.ds(base + c * GC, GC)])

    return _k2_gather


def _k3_body(g6_ref, gd_ref, et_ref, ht_ref, tc_ref, uc_ref, dc_ref,
             wdl_ref, wdh_ref, w1_ref, w2_ref, p_ref):
    et = et_ref[:]
    ht = ht_ref[:]
    zero = jnp.zeros((TE, NB), jnp.float32)

    def sel_coeff(table_ref):
        c = zero
        for t in range(NTY):
            c = c + jnp.where(et == t, table_ref[t:t + 1, :], 0.0)
        return c

    c_text = sel_coeff(tc_ref)
    c_user = sel_coeff(uc_ref)
    c_dst = sel_coeff(dc_ref)
    is_h = ht > 0
    c_edge = jnp.concatenate(
        [jnp.where(is_h, c_text, 0.0), jnp.where(is_h, 0.0, c_user)], axis=1)

    y6lo, y6hi = _unpack_halves(g6_ref[:])
    ef = jnp.zeros((TE, HID), jnp.float32)
    for bi in range(NB):
        ef = ef + c_edge[:, bi:bi + 1] * y6lo[:, bi * HID:(bi + 1) * HID]
    for bi in range(NB, 2 * NB):
        ef = ef + (c_edge[:, bi:bi + 1]
                   * y6hi[:, (bi - NB) * HID:(bi - NB + 1) * HID])
    ef = jnp.maximum(ef, 0.0)

    gdlo, gdhi = _unpack_halves(gd_ref[:])
    ydl = (jnp.dot(gdlo.astype(jnp.bfloat16), wdl_ref[:],
                   preferred_element_type=jnp.float32)
           + jnp.dot(gdhi.astype(jnp.bfloat16), wdh_ref[:],
                     preferred_element_type=jnp.float32))
    yd = jnp.zeros((TE, DIN), jnp.float32)
    for bi in range(NB):
        yd = yd + c_dst[:, bi:bi + 1] * ydl[:, bi * DIN:(bi + 1) * DIN]
    yd = jnp.maximum(yd, 0.0)

    p_ref[0] = jnp.dot(ef.astype(jnp.bfloat16), w1_ref[:],
                       preferred_element_type=jnp.float32)
    p_ref[1] = jnp.dot(yd.astype(jnp.bfloat16), w2_ref[:],
                       preferred_element_type=jnp.float32)


@functools.cache
def _sc_scatter_kernel():
    mesh = plsc.VectorSubcoreMesh(core_axis_name="c", subcore_axis_name="s")

    @functools.partial(
        pl.kernel,
        out_type=jax.ShapeDtypeStruct((NC * ACC_ROWS, HID), jnp.float32),
        mesh=mesh,
        scratch_types=[pltpu.VMEM((CH,), jnp.int32),
                       pltpu.VMEM((CH,), jnp.int32),
                       pltpu.VMEM((CH, HID), jnp.float32),
                       pltpu.VMEM((CH, HID), jnp.float32),
                       pltpu.SemaphoreType.DMA,
                       pltpu.SemaphoreType.DMA,
                       pltpu.MemorySpace.VMEM_SHARED((ACC_ROWS, HID),
                                                     jnp.float32)],
    )
    def _k4_scatter(p_h, i_h, zin_h, out_o, idx0, idx1, rows0, rows1,
                    sem0, sem1, shared):
        cid = lax.axis_index("c")
        sid = lax.axis_index("s")
        wid = sid * NC + cid
        pltpu.sync_copy(zin_h.at[pl.ds(sid * RPT, RPT)],
                        shared.at[pl.ds(sid * RPT, RPT)])
        plsc.subcore_barrier()
        base = wid * RW4
        idxs = (idx0, idx1)
        rows = (rows0, rows1)
        sems = (sem0, sem1)
        cps = [None] * SCH4
        cps[0] = (pltpu.async_copy(i_h.at[pl.ds(base, CH)], idx0, sem0),
                  pltpu.async_copy(p_h.at[pl.ds(base, CH)], rows0, sem0))
        for c in range(SCH4):
            nc = c + 1
            if nc < SCH4:
                off = base + nc * CH
                cps[nc] = (
                    pltpu.async_copy(i_h.at[pl.ds(off, CH)], idxs[nc % 2],
                                     sems[nc % 2]),
                    pltpu.async_copy(p_h.at[pl.ds(off, CH)], rows[nc % 2],
                                     sems[nc % 2]))
            cps[c][0].wait()
            cps[c][1].wait()
            pltpu.sync_copy(rows[c % 2], shared.at[idxs[c % 2]], add=True)
        plsc.subcore_barrier()
        pltpu.sync_copy(shared.at[pl.ds(sid * RPT, RPT)],
                        out_o.at[pl.ds(cid * ACC_ROWS + sid * RPT, RPT)])

    return _k4_scatter


def _k5_body(pa_ref, lb_ref, out_ref):
    z = jnp.maximum(pa_ref[0] + pa_ref[1] + lb_ref[:], 0.0)
    zn = jnp.sqrt(jnp.sum(z * z, axis=-1, keepdims=True))
    zn = jnp.where(zn == 0.0, 1.0, zn)
    out_ref[:] = z / zn


def kernel(h, edge_index, etype, hete, ln_g, ln_b, text_Wb, text_c,
           user_Wb, user_c, dst_Wb, dst_c, lin_W, lin_b):
    src = edge_index[0].astype(jnp.int32)
    dst = edge_index[1].astype(jnp.int32)

    ish = (hete > 0).astype(jnp.int32)
    ch = jnp.cumsum(ish)
    cnh = jnp.cumsum(1 - ish)
    nh = ch[-1]
    pos = jnp.where(ish > 0, ch - 1, nh + cnh - 1)
    tgt = dst[pos]

    pad = EP - NE
    sink = jnp.full((pad,), NN, jnp.int32)
    zpad = jnp.zeros((pad,), jnp.int32)
    src_p = jnp.concatenate([src, zpad])
    dst_p = jnp.concatenate([dst, zpad])
    scat_i = jnp.concatenate([tgt, sink, dst, sink])
    et_p = jnp.concatenate([etype.astype(jnp.int32),
                            jnp.zeros((pad,), jnp.int32)]).reshape(EP, 1)
    ht_p = jnp.concatenate([hete.astype(jnp.int32),
                            jnp.zeros((pad,), jnp.int32)]).reshape(EP, 1)

    w6 = jnp.concatenate([
        jnp.moveaxis(text_Wb, 0, 1).reshape(DIN, NB * HID),
        jnp.moveaxis(user_Wb, 0, 1).reshape(DIN, NB * HID)],
        axis=1).astype(jnp.bfloat16)
    wd = jnp.moveaxis(dst_Wb, 0, 1).reshape(DIN, NB * DIN).astype(jnp.bfloat16)
    wdl = wd[:HD2]
    wdh = wd[HD2:]
    w1 = lin_W[:HID].astype(jnp.bfloat16)
    w2 = lin_W[HID:].astype(jnp.bfloat16)

    hnb, y6n = pl.pallas_call(
        _k1_body,
        grid=(NN // LNC,),
        in_specs=[
            pl.BlockSpec((LNC, DIN), lambda i: (i, 0)),
            pl.BlockSpec((1, DIN), lambda i: (0, 0)),
            pl.BlockSpec((1, DIN), lambda i: (0, 0)),
            pl.BlockSpec((DIN, 2 * NB * HID), lambda i: (0, 0)),
        ],
        out_specs=[pl.BlockSpec((LNC, HD2), lambda i: (i, 0)),
                   pl.BlockSpec((LNC, HD2), lambda i: (i, 0))],
        out_shape=[jax.ShapeDtypeStruct((NN, HD2), jnp.int32),
                   jax.ShapeDtypeStruct((NN, HD2), jnp.int32)],
    )(h, ln_g.reshape(1, DIN), ln_b.reshape(1, DIN), w6)

    g6, gd = _sc_gather_kernel()(y6n, hnb, src_p, dst_p)

    p = pl.pallas_call(
        _k3_body,
        grid=(NT,),
        in_specs=[
            pl.BlockSpec((TE, HD2), lambda i: (i, 0)),
            pl.BlockSpec((TE, HD2), lambda i: (i, 0)),
            pl.BlockSpec((TE, 1), lambda i: (i, 0)),
            pl.BlockSpec((TE, 1), lambda i: (i, 0)),
            pl.BlockSpec((NTY, NB), lambda i: (0, 0)),
            pl.BlockSpec((NTY, NB), lambda i: (0, 0)),
            pl.BlockSpec((NTY, NB), lambda i: (0, 0)),
            pl.BlockSpec((HD2, NB * DIN), lambda i: (0, 0)),
            pl.BlockSpec((HD2, NB * DIN), lambda i: (0, 0)),
            pl.BlockSpec((HID, DOUT), lambda i: (0, 0)),
            pl.BlockSpec((DIN, DOUT), lambda i: (0, 0)),
        ],
        out_specs=pl.BlockSpec((2, TE, HID), lambda i: (0, i, 0)),
        out_shape=jax.ShapeDtypeStruct((2, EP, HID), jnp.float32),
    )(g6, gd, et_p, ht_p, text_c, user_c, dst_c, wdl, wdh, w1, w2)

    zin = jnp.zeros((ACC_ROWS, HID), jnp.float32)
    part = _sc_scatter_kernel()(p.reshape(2 * EP, HID), scat_i, zin)
    part = part.reshape(NC, ACC_ROWS, HID)

    outp = pl.pallas_call(
        _k5_body,
        grid=(ACC_ROWS // FNC,),
        in_specs=[
            pl.BlockSpec((NC, FNC, HID), lambda i: (0, i, 0)),
            pl.BlockSpec((1, DOUT), lambda i: (0, 0)),
        ],
        out_specs=pl.BlockSpec((FNC, DOUT), lambda i: (i, 0)),
        out_shape=jax.ShapeDtypeStruct((ACC_ROWS, DOUT), jnp.float32),
    )(part, lin_b.reshape(1, DOUT))
    return outp[:NN]


# speedup vs baseline: 2.0638x; 1.0645x over previous
import functools

import jax
import jax.numpy as jnp
from jax import lax
from jax.experimental import pallas as pl
from jax.experimental.pallas import tpu as pltpu
from jax.experimental.pallas import tpu_sc as plsc

NN = 10000
NE = 20000
DIN = 768
HID = 128
DOUT = 128
NTY = 4
NB = 3

TE = 512
EP = 20480
NT = EP // TE
ACC_ROWS = 10240
LNC = 400
FNC = 2048

NC = 2
NS = 16
NW = NC * NS
CH = 128
RPW = EP // NW
GCH = RPW // CH
RW4 = 2 * EP // NW
SCH4 = RW4 // CH
RPT = ACC_ROWS // NS


HD2 = DIN // 2


def _bf16_bits(v):
    b = lax.bitcast_convert_type(v, jnp.uint32)
    rnd = ((b >> 16) & 1) + jnp.uint32(0x7FFF)
    return (b + rnd) >> 16


def _pack_halves(x):
    lo = _bf16_bits(x[:, :HD2])
    hi = _bf16_bits(x[:, HD2:])
    return lax.bitcast_convert_type(lo | (hi << 16), jnp.int32)


def _unpack_halves(w):
    wu = lax.bitcast_convert_type(w, jnp.uint32)
    lo = lax.bitcast_convert_type(wu << 16, jnp.float32)
    hi = lax.bitcast_convert_type(wu & jnp.uint32(0xFFFF0000), jnp.float32)
    return lo, hi


def _k1_body(h_ref, g_ref, b_ref, w6_ref, hnb_ref, y6n_ref):
    x = h_ref[:]
    mu = jnp.mean(x, axis=-1, keepdims=True)
    var = jnp.mean((x - mu) ** 2, axis=-1, keepdims=True)
    ln = (x - mu) / jnp.sqrt(var + 1e-5) * g_ref[:] + b_ref[:]
    hnb_ref[:] = _pack_halves(ln)
    y6n_ref[:] = _pack_halves(jnp.dot(ln.astype(jnp.bfloat16), w6_ref[:],
                                      preferred_element_type=jnp.float32))


GC = 64
NGC = RPW // GC
NBUF = 4


@functools.cache
def _sc_gather_kernel():
    mesh = plsc.VectorSubcoreMesh(core_axis_name="c", subcore_axis_name="s")

    @functools.partial(
        pl.kernel,
        out_type=(jax.ShapeDtypeStruct((EP, HD2), jnp.int32),
                  jax.ShapeDtypeStruct((EP, HD2), jnp.int32)),
        mesh=mesh,
        scratch_types=[pltpu.VMEM((RPW,), jnp.int32)]
                      + [pltpu.VMEM((GC, HD2), jnp.int32)] * NBUF
                      + [pltpu.SemaphoreType.DMA] * NBUF,
    )
    def _k2_gather(y6n_h, hnb_h, srcp_h, dstp_h, g6_o, gd_o, idxf, *bufsems):
        bufs = bufsems[:NBUF]
        sems = bufsems[NBUF:]
        wid = lax.axis_index("s") * NC + lax.axis_index("c")
        base = wid * RPW
        for tbl, idxarr, out in ((y6n_h, srcp_h, g6_o), (hnb_h, dstp_h, gd_o)):
            pltpu.sync_copy(idxarr.at[pl.ds(base, RPW)], idxf)
            cps = [None] * NGC
            for c in range(NBUF - 1):
                cps[c] = pltpu.async_copy(
                    tbl.at[idxf.at[pl.ds(c * GC, GC)]], bufs[c % NBUF],
                    sems[c % NBUF])
            for c in range(NGC):
                nc = c + NBUF - 1
                if nc < NGC:
                    cps[nc] = pltpu.async_copy(
                        tbl.at[idxf.at[pl.ds(nc * GC, GC)]], bufs[nc % NBUF],
                        sems[nc % NBUF])
                cps[c].wait()
                pltpu.sync_copy(bufs[c % NBUF],
                                out.at[pl.ds(base + c * GC, GC)])

    return _k2_gather


def _k3_body(g6_ref, gd_ref, et_ref, ht_ref, tc_ref, uc_ref, dc_ref,
             wdl_ref, wdh_ref, w1_ref, w2_ref, p_ref):
    et = et_ref[:]
    ht = ht_ref[:]
    zero = jnp.zeros((TE, NB), jnp.float32)

    def sel_coeff(table_ref):
        c = zero
        for t in range(NTY):
            c = c + jnp.where(et == t, table_ref[t:t + 1, :], 0.0)
        return c

    c_text = sel_coeff(tc_ref)
    c_user = sel_coeff(uc_ref)
    c_dst = sel_coeff(dc_ref)
    is_h = ht > 0
    c_edge = jnp.concatenate(
        [jnp.where(is_h, c_text, 0.0), jnp.where(is_h, 0.0, c_user)], axis=1)

    y6lo, y6hi = _unpack_halves(g6_ref[:])
    ef = jnp.zeros((TE, HID), jnp.float32)
    for bi in range(NB):
        ef = ef + c_edge[:, bi:bi + 1] * y6lo[:, bi * HID:(bi + 1) * HID]
    for bi in range(NB, 2 * NB):
        ef = ef + (c_edge[:, bi:bi + 1]
                   * y6hi[:, (bi - NB) * HID:(bi - NB + 1) * HID])
    ef = jnp.maximum(ef, 0.0)

    gdlo, gdhi = _unpack_halves(gd_ref[:])
    ydl = (jnp.dot(gdlo.astype(jnp.bfloat16), wdl_ref[:],
                   preferred_element_type=jnp.float32)
           + jnp.dot(gdhi.astype(jnp.bfloat16), wdh_ref[:],
                     preferred_element_type=jnp.float32))
    yd = jnp.zeros((TE, DIN), jnp.float32)
    for bi in range(NB):
        yd = yd + c_dst[:, bi:bi + 1] * ydl[:, bi * DIN:(bi + 1) * DIN]
    yd = jnp.maximum(yd, 0.0)

    p_ref[0] = jnp.dot(ef.astype(jnp.bfloat16), w1_ref[:],
                       preferred_element_type=jnp.float32)
    p_ref[1] = jnp.dot(yd.astype(jnp.bfloat16), w2_ref[:],
                       preferred_element_type=jnp.float32)


@functools.cache
def _sc_scatter_kernel():
    mesh = plsc.VectorSubcoreMesh(core_axis_name="c", subcore_axis_name="s")

    @functools.partial(
        pl.kernel,
        out_type=jax.ShapeDtypeStruct((NC * ACC_ROWS, HID), jnp.float32),
        mesh=mesh,
        scratch_types=[pltpu.VMEM((CH,), jnp.int32),
                       pltpu.VMEM((CH,), jnp.int32),
                       pltpu.VMEM((CH, HID), jnp.float32),
                       pltpu.VMEM((CH, HID), jnp.float32),
                       pltpu.SemaphoreType.DMA,
                       pltpu.SemaphoreType.DMA,
                       pltpu.MemorySpace.VMEM_SHARED((ACC_ROWS, HID),
                                                     jnp.float32)],
    )
    def _k4_scatter(p_h, i_h, zin_h, out_o, idx0, idx1, rows0, rows1,
                    sem0, sem1, shared):
        cid = lax.axis_index("c")
        sid = lax.axis_index("s")
        wid = sid * NC + cid
        pltpu.sync_copy(zin_h.at[pl.ds(sid * RPT, RPT)],
                        shared.at[pl.ds(sid * RPT, RPT)])
        plsc.subcore_barrier()
        base = wid * RW4
        idxs = (idx0, idx1)
        rows = (rows0, rows1)
        sems = (sem0, sem1)
        cps = [None] * SCH4
        cps[0] = (pltpu.async_copy(i_h.at[pl.ds(base, CH)], idx0, sem0),
                  pltpu.async_copy(p_h.at[pl.ds(base, CH)], rows0, sem0))
        for c in range(SCH4):
            nc = c + 1
            if nc < SCH4:
                off = base + nc * CH
                cps[nc] = (
                    pltpu.async_copy(i_h.at[pl.ds(off, CH)], idxs[nc % 2],
                                     sems[nc % 2]),
                    pltpu.async_copy(p_h.at[pl.ds(off, CH)], rows[nc % 2],
                                     sems[nc % 2]))
            cps[c][0].wait()
            cps[c][1].wait()
            pltpu.sync_copy(rows[c % 2], shared.at[idxs[c % 2]], add=True)
        plsc.subcore_barrier()
        pltpu.sync_copy(shared.at[pl.ds(sid * RPT, RPT)],
                        out_o.at[pl.ds(cid * ACC_ROWS + sid * RPT, RPT)])

    return _k4_scatter


def _k5_body(pa_ref, lb_ref, out_ref):
    z = jnp.maximum(pa_ref[0] + pa_ref[1] + lb_ref[:], 0.0)
    zn = jnp.sqrt(jnp.sum(z * z, axis=-1, keepdims=True))
    zn = jnp.where(zn == 0.0, 1.0, zn)
    out_ref[:] = z / zn


def kernel(h, edge_index, etype, hete, ln_g, ln_b, text_Wb, text_c,
           user_Wb, user_c, dst_Wb, dst_c, lin_W, lin_b):
    src = edge_index[0].astype(jnp.int32)
    dst = edge_index[1].astype(jnp.int32)

    ish = (hete > 0).astype(jnp.int32)
    ch = jnp.cumsum(ish)
    cnh = jnp.cumsum(1 - ish)
    nh = ch[-1]
    pos = jnp.where(ish > 0, ch - 1, nh + cnh - 1)
    tgt = dst[pos]

    pad = EP - NE
    sink = jnp.full((pad,), NN, jnp.int32)
    zpad = jnp.zeros((pad,), jnp.int32)
    src_p = jnp.concatenate([src, zpad])
    dst_p = jnp.concatenate([dst, zpad])
    scat_i = jnp.concatenate([tgt, sink, dst, sink])
    et_p = jnp.concatenate([etype.astype(jnp.int32),
                            jnp.zeros((pad,), jnp.int32)]).reshape(EP, 1)
    ht_p = jnp.concatenate([hete.astype(jnp.int32),
                            jnp.zeros((pad,), jnp.int32)]).reshape(EP, 1)

    w6 = jnp.concatenate([
        jnp.moveaxis(text_Wb, 0, 1).reshape(DIN, NB * HID),
        jnp.moveaxis(user_Wb, 0, 1).reshape(DIN, NB * HID)],
        axis=1).astype(jnp.bfloat16)
    wd = jnp.moveaxis(dst_Wb, 0, 1).reshape(DIN, NB * DIN).astype(jnp.bfloat16)
    wdl = wd[:HD2]
    wdh = wd[HD2:]
    w1 = lin_W[:HID].astype(jnp.bfloat16)
    w2 = lin_W[HID:].astype(jnp.bfloat16)

    hnb, y6n = pl.pallas_call(
        _k1_body,
        grid=(NN // LNC,),
        in_specs=[
            pl.BlockSpec((LNC, DIN), lambda i: (i, 0)),
            pl.BlockSpec((1, DIN), lambda i: (0, 0)),
            pl.BlockSpec((1, DIN), lambda i: (0, 0)),
            pl.BlockSpec((DIN, 2 * NB * HID), lambda i: (0, 0)),
        ],
        out_specs=[pl.BlockSpec((LNC, HD2), lambda i: (i, 0)),
                   pl.BlockSpec((LNC, HD2), lambda i: (i, 0))],
        out_shape=[jax.ShapeDtypeStruct((NN, HD2), jnp.int32),
                   jax.ShapeDtypeStruct((NN, HD2), jnp.int32)],
    )(h, ln_g.reshape(1, DIN), ln_b.reshape(1, DIN), w6)

    g6, gd = _sc_gather_kernel()(y6n, hnb, src_p, dst_p)

    p = pl.pallas_call(
        _k3_body,
        grid=(NT,),
        in_specs=[
            pl.BlockSpec((TE, HD2), lambda i: (i, 0)),
            pl.BlockSpec((TE, HD2), lambda i: (i, 0)),
            pl.BlockSpec((TE, 1), lambda i: (i, 0)),
            pl.BlockSpec((TE, 1), lambda i: (i, 0)),
            pl.BlockSpec((NTY, NB), lambda i: (0, 0)),
            pl.BlockSpec((NTY, NB), lambda i: (0, 0)),
            pl.BlockSpec((NTY, NB), lambda i: (0, 0)),
            pl.BlockSpec((HD2, NB * DIN), lambda i: (0, 0)),
            pl.BlockSpec((HD2, NB * DIN), lambda i: (0, 0)),
            pl.BlockSpec((HID, DOUT), lambda i: (0, 0)),
            pl.BlockSpec((DIN, DOUT), lambda i: (0, 0)),
        ],
        out_specs=pl.BlockSpec((2, TE, HID), lambda i: (0, i, 0)),
        out_shape=jax.ShapeDtypeStruct((2, EP, HID), jnp.float32),
    )(g6, gd, et_p, ht_p, text_c, user_c, dst_c, wdl, wdh, w1, w2)

    zin = jnp.zeros((ACC_ROWS, HID), jnp.float32)
    part = _sc_scatter_kernel()(p.reshape(2 * EP, HID), scat_i, zin)
    part = part.reshape(NC, ACC_ROWS, HID)

    outp = pl.pallas_call(
        _k5_body,
        grid=(ACC_ROWS // FNC,),
        in_specs=[
            pl.BlockSpec((NC, FNC, HID), lambda i: (0, i, 0)),
            pl.BlockSpec((1, DOUT), lambda i: (0, 0)),
        ],
        out_specs=pl.BlockSpec((FNC, DOUT), lambda i: (i, 0)),
        out_shape=jax.ShapeDtypeStruct((ACC_ROWS, DOUT), jnp.float32),
    )(part, lin_b.reshape(1, DOUT))
    return outp[:NN]


# speedup vs baseline: 2.0911x; 1.0132x over previous
import functools

import jax
import jax.numpy as jnp
from jax import lax
from jax.experimental import pallas as pl
from jax.experimental.pallas import tpu as pltpu
from jax.experimental.pallas import tpu_sc as plsc

NN = 10000
NE = 20000
DIN = 768
HID = 128
DOUT = 128
NTY = 4
NB = 3

TE = 1024
EP = 20480
NT = EP // TE
ACC_ROWS = 10240
LNC = 400
FNC = 2048

NC = 2
NS = 16
NW = NC * NS
CH = 128
RPW = EP // NW
GCH = RPW // CH
RW4 = 2 * EP // NW
SCH4 = RW4 // CH
RPT = ACC_ROWS // NS


HD2 = DIN // 2


def _bf16_bits(v):
    b = lax.bitcast_convert_type(v, jnp.uint32)
    rnd = ((b >> 16) & 1) + jnp.uint32(0x7FFF)
    return (b + rnd) >> 16


def _pack_halves(x):
    lo = _bf16_bits(x[:, :HD2])
    hi = _bf16_bits(x[:, HD2:])
    return lax.bitcast_convert_type(lo | (hi << 16), jnp.int32)


def _unpack_halves(w):
    wu = lax.bitcast_convert_type(w, jnp.uint32)
    lo = lax.bitcast_convert_type(wu << 16, jnp.float32)
    hi = lax.bitcast_convert_type(wu & jnp.uint32(0xFFFF0000), jnp.float32)
    return lo, hi


def _k1_body(h_ref, g_ref, b_ref, w6_ref, hnb_ref, y6n_ref):
    x = h_ref[:]
    mu = jnp.mean(x, axis=-1, keepdims=True)
    var = jnp.mean((x - mu) ** 2, axis=-1, keepdims=True)
    ln = (x - mu) / jnp.sqrt(var + 1e-5) * g_ref[:] + b_ref[:]
    hnb_ref[:] = _pack_halves(ln)
    y6n_ref[:] = _pack_halves(jnp.dot(ln.astype(jnp.bfloat16), w6_ref[:],
                                      preferred_element_type=jnp.float32))


GC = 64
NGC = RPW // GC
NBUF = 4


@functools.cache
def _sc_gather_kernel():
    mesh = plsc.VectorSubcoreMesh(core_axis_name="c", subcore_axis_name="s")

    @functools.partial(
        pl.kernel,
        out_type=(jax.ShapeDtypeStruct((EP, HD2), jnp.int32),
                  jax.ShapeDtypeStruct((EP, HD2), jnp.int32)),
        mesh=mesh,
        scratch_types=[pltpu.VMEM((RPW,), jnp.int32)]
                      + [pltpu.VMEM((GC, HD2), jnp.int32)] * NBUF
                      + [pltpu.SemaphoreType.DMA] * NBUF,
    )
    def _k2_gather(y6n_h, hnb_h, srcp_h, dstp_h, g6_o, gd_o, idxf, *bufsems):
        bufs = bufsems[:NBUF]
        sems = bufsems[NBUF:]
        wid = lax.axis_index("s") * NC + lax.axis_index("c")
        base = wid * RPW
        for tbl, idxarr, out in ((y6n_h, srcp_h, g6_o), (hnb_h, dstp_h, gd_o)):
            pltpu.sync_copy(idxarr.at[pl.ds(base, RPW)], idxf)
            cps = [None] * NGC
            for c in range(NBUF - 1):
                cps[c] = pltpu.async_copy(
                    tbl.at[idxf.at[pl.ds(c * GC, GC)]], bufs[c % NBUF],
                    sems[c % NBUF])
            for c in range(NGC):
                nc = c + NBUF - 1
                if nc < NGC:
                    cps[nc] = pltpu.async_copy(
                        tbl.at[idxf.at[pl.ds(nc * GC, GC)]], bufs[nc % NBUF],
                        sems[nc % NBUF])
                cps[c].wait()
                pltpu.sync_copy(bufs[c % NBUF],
                                out.at[pl.ds(base + c * GC, GC)])

    return _k2_gather


def _k3_body(g6_ref, gd_ref, et_ref, ht_ref, tc_ref, uc_ref, dc_ref,
             wdl_ref, wdh_ref, w1_ref, w2_ref, p_ref):
    et = et_ref[:]
    ht = ht_ref[:]
    zero = jnp.zeros((TE, NB), jnp.float32)

    def sel_coeff(table_ref):
        c = zero
        for t in range(NTY):
            c = c + jnp.where(et == t, table_ref[t:t + 1, :], 0.0)
        return c

    c_text = sel_coeff(tc_ref)
    c_user = sel_coeff(uc_ref)
    c_dst = sel_coeff(dc_ref)
    is_h = ht > 0
    c_edge = jnp.concatenate(
        [jnp.where(is_h, c_text, 0.0), jnp.where(is_h, 0.0, c_user)], axis=1)

    y6lo, y6hi = _unpack_halves(g6_ref[:])
    ef = jnp.zeros((TE, HID), jnp.float32)
    for bi in range(NB):
        ef = ef + c_edge[:, bi:bi + 1] * y6lo[:, bi * HID:(bi + 1) * HID]
    for bi in range(NB, 2 * NB):
        ef = ef + (c_edge[:, bi:bi + 1]
                   * y6hi[:, (bi - NB) * HID:(bi - NB + 1) * HID])
    ef = jnp.maximum(ef, 0.0)

    gdlo, gdhi = _unpack_halves(gd_ref[:])
    ydl = (jnp.dot(gdlo.astype(jnp.bfloat16), wdl_ref[:],
                   preferred_element_type=jnp.float32)
           + jnp.dot(gdhi.astype(jnp.bfloat16), wdh_ref[:],
                     preferred_element_type=jnp.float32))
    yd = jnp.zeros((TE, DIN), jnp.float32)
    for bi in range(NB):
        yd = yd + c_dst[:, bi:bi + 1] * ydl[:, bi * DIN:(bi + 1) * DIN]
    yd = jnp.maximum(yd, 0.0)

    p_ref[0] = jnp.dot(ef.astype(jnp.bfloat16), w1_ref[:],
                       preferred_element_type=jnp.float32)
    p_ref[1] = jnp.dot(yd.astype(jnp.bfloat16), w2_ref[:],
                       preferred_element_type=jnp.float32)


@functools.cache
def _sc_scatter_kernel():
    mesh = plsc.VectorSubcoreMesh(core_axis_name="c", subcore_axis_name="s")

    @functools.partial(
        pl.kernel,
        out_type=jax.ShapeDtypeStruct((NC * ACC_ROWS, HID), jnp.float32),
        mesh=mesh,
        scratch_types=[pltpu.VMEM((CH,), jnp.int32),
                       pltpu.VMEM((CH,), jnp.int32),
                       pltpu.VMEM((CH, HID), jnp.float32),
                       pltpu.VMEM((CH, HID), jnp.float32),
                       pltpu.SemaphoreType.DMA,
                       pltpu.SemaphoreType.DMA,
                       pltpu.MemorySpace.VMEM_SHARED((ACC_ROWS, HID),
                                                     jnp.float32)],
    )
    def _k4_scatter(p_h, i_h, zin_h, out_o, idx0, idx1, rows0, rows1,
                    sem0, sem1, shared):
        cid = lax.axis_index("c")
        sid = lax.axis_index("s")
        wid = sid * NC + cid
        pltpu.sync_copy(zin_h.at[pl.ds(sid * RPT, RPT)],
                        shared.at[pl.ds(sid * RPT, RPT)])
        plsc.subcore_barrier()
        base = wid * RW4
        idxs = (idx0, idx1)
        rows = (rows0, rows1)
        sems = (sem0, sem1)
        cps = [None] * SCH4
        cps[0] = (pltpu.async_copy(i_h.at[pl.ds(base, CH)], idx0, sem0),
                  pltpu.async_copy(p_h.at[pl.ds(base, CH)], rows0, sem0))
        for c in range(SCH4):
            nc = c + 1
            if nc < SCH4:
                off = base + nc * CH
                cps[nc] = (
                    pltpu.async_copy(i_h.at[pl.ds(off, CH)], idxs[nc % 2],
                                     sems[nc % 2]),
                    pltpu.async_copy(p_h.at[pl.ds(off, CH)], rows[nc % 2],
                                     sems[nc % 2]))
            cps[c][0].wait()
            cps[c][1].wait()
            pltpu.sync_copy(rows[c % 2], shared.at[idxs[c % 2]], add=True)
        plsc.subcore_barrier()
        pltpu.sync_copy(shared.at[pl.ds(sid * RPT, RPT)],
                        out_o.at[pl.ds(cid * ACC_ROWS + sid * RPT, RPT)])

    return _k4_scatter


def _k5_body(pa_ref, lb_ref, out_ref):
    z = jnp.maximum(pa_ref[0] + pa_ref[1] + lb_ref[:], 0.0)
    zn = jnp.sqrt(jnp.sum(z * z, axis=-1, keepdims=True))
    zn = jnp.where(zn == 0.0, 1.0, zn)
    out_ref[:] = z / zn


def kernel(h, edge_index, etype, hete, ln_g, ln_b, text_Wb, text_c,
           user_Wb, user_c, dst_Wb, dst_c, lin_W, lin_b):
    src = edge_index[0].astype(jnp.int32)
    dst = edge_index[1].astype(jnp.int32)

    ish = (hete > 0).astype(jnp.int32)
    ch = jnp.cumsum(ish)
    cnh = jnp.cumsum(1 - ish)
    nh = ch[-1]
    pos = jnp.where(ish > 0, ch - 1, nh + cnh - 1)
    tgt = dst[pos]

    pad = EP - NE
    sink = jnp.full((pad,), NN, jnp.int32)
    zpad = jnp.zeros((pad,), jnp.int32)
    src_p = jnp.concatenate([src, zpad])
    dst_p = jnp.concatenate([dst, zpad])
    scat_i = jnp.concatenate([tgt, sink, dst, sink])
    et_p = jnp.concatenate([etype.astype(jnp.int32),
                            jnp.zeros((pad,), jnp.int32)]).reshape(EP, 1)
    ht_p = jnp.concatenate([hete.astype(jnp.int32),
                            jnp.zeros((pad,), jnp.int32)]).reshape(EP, 1)

    w6 = jnp.concatenate([
        jnp.moveaxis(text_Wb, 0, 1).reshape(DIN, NB * HID),
        jnp.moveaxis(user_Wb, 0, 1).reshape(DIN, NB * HID)],
        axis=1).astype(jnp.bfloat16)
    wd = jnp.moveaxis(dst_Wb, 0, 1).reshape(DIN, NB * DIN).astype(jnp.bfloat16)
    wdl = wd[:HD2]
    wdh = wd[HD2:]
    w1 = lin_W[:HID].astype(jnp.bfloat16)
    w2 = lin_W[HID:].astype(jnp.bfloat16)

    hnb, y6n = pl.pallas_call(
        _k1_body,
        grid=(NN // LNC,),
        in_specs=[
            pl.BlockSpec((LNC, DIN), lambda i: (i, 0)),
            pl.BlockSpec((1, DIN), lambda i: (0, 0)),
            pl.BlockSpec((1, DIN), lambda i: (0, 0)),
            pl.BlockSpec((DIN, 2 * NB * HID), lambda i: (0, 0)),
        ],
        out_specs=[pl.BlockSpec((LNC, HD2), lambda i: (i, 0)),
                   pl.BlockSpec((LNC, HD2), lambda i: (i, 0))],
        out_shape=[jax.ShapeDtypeStruct((NN, HD2), jnp.int32),
                   jax.ShapeDtypeStruct((NN, HD2), jnp.int32)],
    )(h, ln_g.reshape(1, DIN), ln_b.reshape(1, DIN), w6)

    g6, gd = _sc_gather_kernel()(y6n, hnb, src_p, dst_p)

    p = pl.pallas_call(
        _k3_body,
        grid=(NT,),
        in_specs=[
            pl.BlockSpec((TE, HD2), lambda i: (i, 0)),
            pl.BlockSpec((TE, HD2), lambda i: (i, 0)),
            pl.BlockSpec((TE, 1), lambda i: (i, 0)),
            pl.BlockSpec((TE, 1), lambda i: (i, 0)),
            pl.BlockSpec((NTY, NB), lambda i: (0, 0)),
            pl.BlockSpec((NTY, NB), lambda i: (0, 0)),
            pl.BlockSpec((NTY, NB), lambda i: (0, 0)),
            pl.BlockSpec((HD2, NB * DIN), lambda i: (0, 0)),
            pl.BlockSpec((HD2, NB * DIN), lambda i: (0, 0)),
            pl.BlockSpec((HID, DOUT), lambda i: (0, 0)),
            pl.BlockSpec((DIN, DOUT), lambda i: (0, 0)),
        ],
        out_specs=pl.BlockSpec((2, TE, HID), lambda i: (0, i, 0)),
        out_shape=jax.ShapeDtypeStruct((2, EP, HID), jnp.float32),
    )(g6, gd, et_p, ht_p, text_c, user_c, dst_c, wdl, wdh, w1, w2)

    zin = jnp.zeros((ACC_ROWS, HID), jnp.float32)
    part = _sc_scatter_kernel()(p.reshape(2 * EP, HID), scat_i, zin)
    part = part.reshape(NC, ACC_ROWS, HID)

    outp = pl.pallas_call(
        _k5_body,
        grid=(ACC_ROWS // FNC,),
        in_specs=[
            pl.BlockSpec((NC, FNC, HID), lambda i: (0, i, 0)),
            pl.BlockSpec((1, DOUT), lambda i: (0, 0)),
        ],
        out_specs=pl.BlockSpec((FNC, DOUT), lambda i: (i, 0)),
        out_shape=jax.ShapeDtypeStruct((ACC_ROWS, DOUT), jnp.float32),
    )(part, lin_b.reshape(1, DOUT))
    return outp[:NN]
